# baseline (speedup 1.0000x reference)
import functools

import jax
import jax.numpy as jnp
from jax import lax
from jax.experimental import pallas as pl
from jax.experimental.pallas import tpu as pltpu
from jax.experimental.pallas import tpu_sc as plsc

_NC = 2
_NS = 16
_NW = _NC * _NS
_L = 16
_Z = (1, 6, 7, 8)


@functools.lru_cache(maxsize=None)
def _sc_kernel(B, A, B_sc):
    MPW = B_sc // _NW
    MB = 128
    NCH = MPW // MB
    NG = MB // _L
    UNROLL = 2
    NFULL = A // UNROLL
    mesh = plsc.VectorSubcoreMesh(core_axis_name="c", subcore_axis_name="s")

    @functools.partial(
        pl.kernel,
        mesh=mesh,
        out_type=jax.ShapeDtypeStruct((B_sc,), jnp.float32),
        compiler_params=pltpu.CompilerParams(
            needs_layout_passes=False,
            skip_device_barrier=True,
            disable_bounds_checks=True,
        ),
        scratch_types=[
            pltpu.VMEM((A, MB), jnp.int32),
            pltpu.VMEM((A, MB), jnp.int32),
            pltpu.VMEM((MPW,), jnp.float32),
            pltpu.VMEM((MPW,), jnp.float32),
            pltpu.VMEM((_L,), jnp.float32),
            pltpu.SemaphoreType.DMA,
            pltpu.SemaphoreType.DMA,
        ],
    )
    def k(sp_hbm, en_hbm, se_hbm, out_hbm, sp_a, sp_b, en_v, out_v, se_v,
          sem_a, sem_b):
        wid = lax.axis_index("s") * _NC + lax.axis_index("c")
        col0 = wid * MPW
        lanes = lax.iota(jnp.int32, _L)
        zero = jnp.zeros((_L,), jnp.float32)
        dnums = lax.GatherDimensionNumbers(
            offset_dims=(), collapsed_slice_dims=(0,), start_index_map=(0,))

        def vgather(vec, idx):
            return lax.gather(vec, idx[:, None], dnums, (1,),
                              mode=lax.GatherScatterMode.PROMISE_IN_BOUNDS)

        se_v[...] = zero
        pltpu.sync_copy(se_hbm, se_v.at[pl.ds(0, len(_Z))])
        nz = len(_Z)
        tmap = jnp.full((_L,), nz, jnp.int32)
        for i, z in enumerate(_Z):
            tmap = jnp.where(lanes == z, jnp.full((_L,), i, jnp.int32), tmap)
        tbl = vgather(se_v[...], tmap)

        pltpu.sync_copy(en_hbm.at[pl.ds(col0, MPW)], en_v)

        def lookup(z):
            zu = jnp.minimum(plsc.bitcast(z, jnp.uint32), jnp.uint32(_L - 1))
            return vgather(tbl, plsc.bitcast(zu, jnp.int32))

        def compute(sp_v, ci):
            def abody(i, accs):
                accs = list(accs)
                for u in range(UNROLL):
                    a = i * UNROLL + u
                    for m in range(NG):
                        accs[m] = accs[m] + lookup(sp_v[a, pl.ds(m * _L, _L)])
                return tuple(accs)

            accs = lax.fori_loop(0, NFULL, abody, (zero,) * NG)
            accs = list(accs)
            for a in range(NFULL * UNROLL, A):
                for m in range(NG):
                    accs[m] = accs[m] + lookup(sp_v[a, pl.ds(m * _L, _L)])
            base = ci * MB
            for m in range(NG):
                sl = pl.ds(base + m * _L, _L)
                out_v[sl] = en_v[sl] - accs[m]

        def sp_src(ci):
            return sp_hbm.at[:, pl.ds(col0 + ci * MB, MB)]

        bufs = ((sp_a, sem_a), (sp_b, sem_b))
        pltpu.async_copy(sp_src(0), sp_a, sem_a)
        for ci in range(NCH):
            buf, sem = bufs[ci % 2]
            if ci + 1 < NCH:
                nbuf, nsem = bufs[(ci + 1) % 2]
                pltpu.async_copy(sp_src(ci + 1), nbuf, nsem)
            pltpu.make_async_copy(sp_src(0), buf, sem).wait()
            compute(buf, ci)
        pltpu.sync_copy(out_v, out_hbm.at[pl.ds(col0, MPW)])

    return k


@functools.lru_cache(maxsize=None)
def _tc_kernel(B, A, B_sc, BC=4096):
    B_tc = B - B_sc
    off = B_sc // BC

    def body(se_ref, sp_ref, en_ref, out_ref):
        z = sp_ref[...]
        tot = jnp.float32(0.0)
        for i, zk in enumerate(_Z):
            tot = jnp.where(z == zk, se_ref[i], tot)
        ones = jnp.ones((1, A), jnp.float32)
        s = lax.dot_general(ones, tot, (((1,), (0,)), ((), ())),
                            precision=lax.Precision.HIGHEST,
                            preferred_element_type=jnp.float32)
        out_ref[...] = en_ref[...] - s[0]

    return pl.pallas_call(
        body,
        grid=(B_tc // BC,),
        in_specs=[
            pl.BlockSpec(memory_space=pltpu.SMEM),
            pl.BlockSpec((A, BC), lambda i: (0, off + i)),
            pl.BlockSpec((BC,), lambda i: (off + i,)),
        ],
        out_specs=pl.BlockSpec((BC,), lambda i: (off + i,)),
        out_shape=jax.ShapeDtypeStruct((B,), jnp.float32),
    )


def kernel(species, energies, self_energies):
    B, A = species.shape
    sp_t = jnp.asarray(species, jnp.int32).T
    en = jnp.asarray(energies, jnp.float32)
    se = jnp.asarray(self_energies, jnp.float32)
    B_sc = B // 4
    sc_out = _sc_kernel(B, A, B_sc)(sp_t, en, se)
    tc_out = _tc_kernel(B, A, B_sc)(se, sp_t, en)
    return lax.dynamic_update_slice(tc_out, sc_out, (0,))


# speedup vs baseline: 1.0255x; 1.0255x over previous
import functools

import jax
import jax.numpy as jnp
from jax import lax
from jax.experimental import pallas as pl
from jax.experimental.pallas import tpu as pltpu
from jax.experimental.pallas import tpu_sc as plsc

_NC = 2
_NS = 16
_NW = _NC * _NS
_L = 16
_Z = (1, 6, 7, 8)


@functools.lru_cache(maxsize=None)
def _sc_kernel(B, A, B_sc):
    MPW = B_sc // _NW
    MB = 128
    NCH = MPW // MB
    NG = MB // _L
    UNROLL = 2
    NFULL = A // UNROLL
    mesh = plsc.VectorSubcoreMesh(core_axis_name="c", subcore_axis_name="s")

    @functools.partial(
        pl.kernel,
        mesh=mesh,
        out_type=jax.ShapeDtypeStruct((B_sc,), jnp.float32),
        compiler_params=pltpu.CompilerParams(
            needs_layout_passes=False,
            skip_device_barrier=True,
            disable_bounds_checks=True,
        ),
        scratch_types=[
            pltpu.VMEM((A, MB), jnp.int32),
            pltpu.VMEM((A, MB), jnp.int32),
            pltpu.VMEM((MPW,), jnp.float32),
            pltpu.VMEM((MPW,), jnp.float32),
            pltpu.VMEM((_L,), jnp.float32),
            pltpu.SemaphoreType.DMA,
            pltpu.SemaphoreType.DMA,
        ],
    )
    def k(sp_hbm, en_hbm, se_hbm, out_hbm, sp_a, sp_b, en_v, out_v, se_v,
          sem_a, sem_b):
        wid = lax.axis_index("s") * _NC + lax.axis_index("c")
        col0 = wid * MPW
        lanes = lax.iota(jnp.int32, _L)
        zero = jnp.zeros((_L,), jnp.float32)
        dnums = lax.GatherDimensionNumbers(
            offset_dims=(), collapsed_slice_dims=(0,), start_index_map=(0,))

        def vgather(vec, idx):
            return lax.gather(vec, idx[:, None], dnums, (1,),
                              mode=lax.GatherScatterMode.PROMISE_IN_BOUNDS)

        se_v[...] = zero
        pltpu.sync_copy(se_hbm, se_v.at[pl.ds(0, len(_Z))])
        nz = len(_Z)
        tmap = jnp.full((_L,), nz, jnp.int32)
        for i, z in enumerate(_Z):
            tmap = jnp.where(lanes == z, jnp.full((_L,), i, jnp.int32), tmap)
        tbl = vgather(se_v[...], tmap)

        pltpu.sync_copy(en_hbm.at[pl.ds(col0, MPW)], en_v)

        def lookup(z):
            zu = jnp.minimum(plsc.bitcast(z, jnp.uint32), jnp.uint32(_L - 1))
            return vgather(tbl, plsc.bitcast(zu, jnp.int32))

        def compute(sp_v, ci):
            def abody(i, accs):
                accs = list(accs)
                for u in range(UNROLL):
                    a = i * UNROLL + u
                    for m in range(NG):
                        accs[m] = accs[m] + lookup(sp_v[a, pl.ds(m * _L, _L)])
                return tuple(accs)

            accs = lax.fori_loop(0, NFULL, abody, (zero,) * NG)
            accs = list(accs)
            for a in range(NFULL * UNROLL, A):
                for m in range(NG):
                    accs[m] = accs[m] + lookup(sp_v[a, pl.ds(m * _L, _L)])
            base = ci * MB
            for m in range(NG):
                sl = pl.ds(base + m * _L, _L)
                out_v[sl] = en_v[sl] - accs[m]

        def sp_src(ci):
            return sp_hbm.at[:, pl.ds(col0 + ci * MB, MB)]

        bufs = ((sp_a, sem_a), (sp_b, sem_b))
        pltpu.async_copy(sp_src(0), sp_a, sem_a)
        for ci in range(NCH):
            buf, sem = bufs[ci % 2]
            if ci + 1 < NCH:
                nbuf, nsem = bufs[(ci + 1) % 2]
                pltpu.async_copy(sp_src(ci + 1), nbuf, nsem)
            pltpu.make_async_copy(sp_src(0), buf, sem).wait()
            compute(buf, ci)
        pltpu.sync_copy(out_v, out_hbm.at[pl.ds(col0, MPW)])

    return k


@functools.lru_cache(maxsize=None)
def _tc_kernel(B, A, B_sc, BC=4096):
    B_tc = B - B_sc
    off = B_sc // BC

    def body(se_ref, sp_ref, en_ref, out_ref):
        z = sp_ref[...]
        tot = jnp.float32(0.0)
        for i, zk in enumerate(_Z):
            tot = jnp.where(z == zk, se_ref[i], tot)
        out_ref[...] = en_ref[...] - jnp.sum(tot, axis=0)

    return pl.pallas_call(
        body,
        grid=(B_tc // BC,),
        in_specs=[
            pl.BlockSpec(memory_space=pltpu.SMEM),
            pl.BlockSpec((A, BC), lambda i: (0, off + i)),
            pl.BlockSpec((BC,), lambda i: (off + i,)),
        ],
        out_specs=pl.BlockSpec((BC,), lambda i: (off + i,)),
        out_shape=jax.ShapeDtypeStruct((B,), jnp.float32),
    )


def kernel(species, energies, self_energies):
    B, A = species.shape
    sp_t = jnp.asarray(species, jnp.int32).T
    en = jnp.asarray(energies, jnp.float32)
    se = jnp.asarray(self_energies, jnp.float32)
    B_sc = B // 4
    sc_out = _sc_kernel(B, A, B_sc)(sp_t, en, se)
    tc_out = _tc_kernel(B, A, B_sc)(se, sp_t, en)
    return lax.dynamic_update_slice(tc_out, sc_out, (0,))
